# baseline (speedup 1.0000x reference)
import functools

import jax
import jax.numpy as jnp
from jax import lax
from jax.experimental import pallas as pl
from jax.experimental.pallas import tpu as pltpu
from jax.experimental.pallas import tpu_sc as plsc

N, E, D, DE, H = 10000, 320000, 128, 16, 128

NC, NS = 2, 16
NW = NC * NS
CH = 128
NCHUNK = E // CH
T_FULL = 632
T_LAST = N - (NS - 1) * T_FULL

NB_NODE = 2000
NB_EDGE = 16000


def _pre_body(x_ref, wa_ref, wb_ref, be1_ref, xa_ref, xb_ref):
    xv = x_ref[...]
    xa_ref[...] = (
        jnp.dot(xv, wa_ref[...], preferred_element_type=jnp.float32)
        + be1_ref[...]
    )
    xb_ref[...] = jnp.dot(xv, wb_ref[...], preferred_element_type=jnp.float32)


def _node_tables(x, wa, wb, be1):
    nblk = N // NB_NODE
    full = lambda shape: pl.BlockSpec(shape, lambda i: (0, 0))
    return pl.pallas_call(
        _pre_body,
        grid=(nblk,),
        in_specs=[
            pl.BlockSpec((NB_NODE, D), lambda i: (i, 0)),
            full((D, H)),
            full((D, H)),
            full((1, H)),
        ],
        out_specs=[
            pl.BlockSpec((NB_NODE, H), lambda i: (i, 0)),
            pl.BlockSpec((NB_NODE, H), lambda i: (i, 0)),
        ],
        out_shape=[
            jax.ShapeDtypeStruct((N, H), jnp.float32),
            jax.ShapeDtypeStruct((N, H), jnp.float32),
        ],
    )(x, wa, wb, be1)


CPW = NCHUNK // NW
NTAIL = NCHUNK - CPW * NW

NIT_G = CPW // 2


def _gather_body(xa_hbm, xb_hbm, row_hbm, col_hbm, g_hbm,
                 idx_r, idx_c, abuf, bbuf, obuf,
                 sg0, sg1, si0, si1, sw0, sw1):
    sg = [sg0, sg1]
    si = [si0, si1]
    sw = [sw0, sw1]
    wid = lax.axis_index("s") * NC + lax.axis_index("c")
    start = wid * CPW

    for b in range(2):
        c0 = start + b
        pltpu.sync_copy(row_hbm.at[pl.ds(c0 * CH, CH)], idx_r.at[b])
        pltpu.sync_copy(col_hbm.at[pl.ds(c0 * CH, CH)], idx_c.at[b])
        pltpu.async_copy(xa_hbm.at[idx_r.at[b]], abuf.at[b], sg[b])
        pltpu.async_copy(xb_hbm.at[idx_c.at[b]], bbuf.at[b], sg[b])

    def slot(i, b):
        c = start + 2 * i + b
        pltpu.make_async_copy(
            xa_hbm.at[idx_r.at[b]], abuf.at[b], sg[b]).wait()
        pltpu.make_async_copy(
            xb_hbm.at[idx_c.at[b]], bbuf.at[b], sg[b]).wait()

        @pl.when(i < NIT_G - 1)
        def _():
            cn = c + 2
            pltpu.async_copy(row_hbm.at[pl.ds(cn * CH, CH)], idx_r.at[b],
                             si[b])
            pltpu.async_copy(col_hbm.at[pl.ds(cn * CH, CH)], idx_c.at[b],
                             si[b])

        @pl.when(i > 0)
        def _():
            pltpu.make_async_copy(
                obuf.at[b], g_hbm.at[pl.ds((c - 2) * CH, CH)], sw[b]).wait()

        @plsc.parallel_loop(0, CH, 1, unroll=8)
        def _(r):
            for j in range(H // 16):
                sl = pl.ds(j * 16, 16)
                obuf[b, r, sl] = abuf[b, r, sl] + bbuf[b, r, sl]

        pltpu.async_copy(obuf.at[b], g_hbm.at[pl.ds(c * CH, CH)], sw[b])

        @pl.when(i < NIT_G - 1)
        def _():
            cn = c + 2
            pltpu.make_async_copy(row_hbm.at[pl.ds(cn * CH, CH)],
                                  idx_r.at[b], si[b]).wait()
            pltpu.make_async_copy(col_hbm.at[pl.ds(cn * CH, CH)],
                                  idx_c.at[b], si[b]).wait()
            pltpu.async_copy(xa_hbm.at[idx_r.at[b]], abuf.at[b], sg[b])
            pltpu.async_copy(xb_hbm.at[idx_c.at[b]], bbuf.at[b], sg[b])

    def it(i, carry):
        slot(i, 0)
        slot(i, 1)
        return carry

    lax.fori_loop(0, NIT_G, it, 0)

    for b in range(2):
        cl = start + 2 * (NIT_G - 1) + b
        pltpu.make_async_copy(
            obuf.at[b], g_hbm.at[pl.ds(cl * CH, CH)], sw[b]).wait()

    @pl.when(wid < NTAIL)
    def _():
        ct = NW * CPW + wid
        pltpu.sync_copy(row_hbm.at[pl.ds(ct * CH, CH)], idx_r.at[0])
        pltpu.sync_copy(col_hbm.at[pl.ds(ct * CH, CH)], idx_c.at[0])
        pltpu.async_copy(xa_hbm.at[idx_r.at[0]], abuf.at[0], sg[0]).wait()
        pltpu.async_copy(xb_hbm.at[idx_c.at[0]], bbuf.at[0], sg[0]).wait()

        def add_row(r, carry):
            for j in range(H // 16):
                sl = pl.ds(j * 16, 16)
                obuf[0, r, sl] = abuf[0, r, sl] + bbuf[0, r, sl]
            return carry

        lax.fori_loop(0, CH, add_row, 0)
        pltpu.sync_copy(obuf.at[0], g_hbm.at[pl.ds(ct * CH, CH)])


def _gather_add(xa, xb, row, col):
    mesh = plsc.VectorSubcoreMesh(core_axis_name="c", subcore_axis_name="s")
    fn = functools.partial(
        pl.kernel,
        mesh=mesh,
        out_type=jax.ShapeDtypeStruct((E, H), jnp.float32),
        scratch_types=[
            pltpu.VMEM((2, CH), jnp.int32),
            pltpu.VMEM((2, CH), jnp.int32),
            pltpu.VMEM((2, CH, H), jnp.float32),
            pltpu.VMEM((2, CH, H), jnp.float32),
            pltpu.VMEM((2, CH, H), jnp.float32),
            pltpu.SemaphoreType.DMA,
            pltpu.SemaphoreType.DMA,
            pltpu.SemaphoreType.DMA,
            pltpu.SemaphoreType.DMA,
            pltpu.SemaphoreType.DMA,
            pltpu.SemaphoreType.DMA,
        ],
    )(_gather_body)
    return fn(xa, xb, row, col)


def _edge_body(g_ref, ea_ref, wc_ref, w2_ref, b2_ref, ef_ref):
    h = jnp.maximum(
        g_ref[...]
        + jnp.dot(ea_ref[...], wc_ref[...], preferred_element_type=jnp.float32),
        0.0,
    )
    ef_ref[...] = jnp.maximum(
        jnp.dot(h, w2_ref[...], preferred_element_type=jnp.float32)
        + b2_ref[...],
        0.0,
    )


def _edge_mlp(g, ea, wc, w2, b2):
    nblk = E // NB_EDGE
    full = lambda shape: pl.BlockSpec(shape, lambda i: (0, 0))
    return pl.pallas_call(
        _edge_body,
        grid=(nblk,),
        in_specs=[
            pl.BlockSpec((NB_EDGE, H), lambda i: (i, 0)),
            pl.BlockSpec((NB_EDGE, DE), lambda i: (i, 0)),
            full((DE, H)),
            full((H, H)),
            full((1, H)),
        ],
        out_specs=pl.BlockSpec((NB_EDGE, H), lambda i: (i, 0)),
        out_shape=jax.ShapeDtypeStruct((E, H), jnp.float32),
    )(g, ea, wc, w2, b2)


NIT_S = CPW // 3


def _scatter_body(ef_hbm, row3_hbm, row1_hbm, agg_hbm, idx_s, ebuf, accum,
                  sf0, sf1, sf2, ss0, ss1, ss2):
    sf = [sf0, sf1, sf2]
    ss = [ss0, ss1, ss2]
    c_id = lax.axis_index("c")
    s_id = lax.axis_index("s")
    wid = s_id * NC + c_id

    zero = jnp.zeros((16,), jnp.float32)

    def zrow(r, carry):
        for j in range(H // 16):
            ebuf[0, r, pl.ds(j * 16, 16)] = zero
        return carry

    lax.fori_loop(0, CH, zrow, 0)
    base_rows = s_id * T_FULL

    @pl.when(s_id < NS - 1)
    def _():
        for k in range(4):
            pltpu.sync_copy(
                ebuf.at[0, pl.ds(0, CH)],
                accum.at[pl.ds(base_rows + k * CH, CH)],
            )
        pltpu.sync_copy(
            ebuf.at[0, pl.ds(0, T_FULL - 4 * CH)],
            accum.at[pl.ds(base_rows + 4 * CH, T_FULL - 4 * CH)],
        )

    @pl.when(s_id == NS - 1)
    def _():
        for k in range(4):
            pltpu.sync_copy(
                ebuf.at[0, pl.ds(0, CH)],
                accum.at[pl.ds(base_rows + k * CH, CH)],
            )
        pltpu.sync_copy(
            ebuf.at[0, pl.ds(0, T_LAST - 4 * CH)],
            accum.at[pl.ds(base_rows + 4 * CH, T_LAST - 4 * CH)],
        )

    plsc.subcore_barrier()

    start = wid * CPW

    for b in range(3):
        c0 = start + b
        pltpu.async_copy(row1_hbm.at[pl.ds(c0 * CH, CH)], idx_s.at[b], sf[b])
        pltpu.async_copy(ef_hbm.at[pl.ds(c0 * CH, CH)], ebuf.at[b], sf[b])

    def slot(i, b):
        c = start + 3 * i + b
        k = 3 * i + b
        pltpu.make_async_copy(
            row1_hbm.at[pl.ds(c * CH, CH)], idx_s.at[b], sf[b]).wait()
        pltpu.make_async_copy(
            ef_hbm.at[pl.ds(c * CH, CH)], ebuf.at[b], sf[b]).wait()
        pltpu.async_copy(ebuf.at[b], accum.at[idx_s.at[b]], ss[b], add=True)

        br = (b + 1) % 3

        @pl.when((k >= 2) & (k <= CPW - 2))
        def _():
            pltpu.make_async_copy(
                ebuf.at[br], accum.at[idx_s.at[br]], ss[br]).wait()
            cn = c + 1
            pltpu.async_copy(row1_hbm.at[pl.ds(cn * CH, CH)], idx_s.at[br],
                             sf[br])
            pltpu.async_copy(ef_hbm.at[pl.ds(cn * CH, CH)], ebuf.at[br],
                             sf[br])

    def it(i, carry):
        slot(i, 0)
        slot(i, 1)
        slot(i, 2)
        return carry

    lax.fori_loop(0, NIT_S, it, 0)

    for b in range(3):
        pltpu.make_async_copy(
            ebuf.at[b], accum.at[idx_s.at[b]], ss[b]).wait()

    @pl.when(wid < NTAIL)
    def _():
        ct = NW * CPW + wid
        pltpu.sync_copy(row1_hbm.at[pl.ds(ct * CH, CH)], idx_s.at[0])
        pltpu.sync_copy(ef_hbm.at[pl.ds(ct * CH, CH)], ebuf.at[0])
        pltpu.sync_copy(ebuf.at[0], accum.at[idx_s.at[0]], add=True)

    plsc.subcore_barrier()

    @pl.when(s_id < NS - 1)
    def _():
        pltpu.sync_copy(
            accum.at[pl.ds(base_rows, T_FULL)],
            agg_hbm.at[pl.ds(c_id * N + base_rows, T_FULL)],
        )

    @pl.when(s_id == NS - 1)
    def _():
        pltpu.sync_copy(
            accum.at[pl.ds(base_rows, T_LAST)],
            agg_hbm.at[pl.ds(c_id * N + base_rows, T_LAST)],
        )


def _segment_sum(ef, row3, row1):
    mesh = plsc.VectorSubcoreMesh(core_axis_name="c", subcore_axis_name="s")
    fn = functools.partial(
        pl.kernel,
        mesh=mesh,
        out_type=jax.ShapeDtypeStruct((NC * N, H), jnp.float32),
        scratch_types=[
            pltpu.VMEM((3, CH), jnp.int32),
            pltpu.VMEM((3, CH, H), jnp.float32),
            pltpu.VMEM_SHARED((N, H), jnp.float32),
            pltpu.SemaphoreType.DMA,
            pltpu.SemaphoreType.DMA,
            pltpu.SemaphoreType.DMA,
            pltpu.SemaphoreType.DMA,
            pltpu.SemaphoreType.DMA,
            pltpu.SemaphoreType.DMA,
        ],
    )(_scatter_body)
    return fn(ef, row3, row1)


def _node_body(x_ref, a0_ref, a1_ref, w1a_ref, w1b_ref, b1_ref, w2_ref,
               b2_ref, out_ref):
    xv = x_ref[...]
    a = a0_ref[...] + a1_ref[...]
    t = jnp.maximum(
        jnp.dot(xv, w1a_ref[...], preferred_element_type=jnp.float32)
        + jnp.dot(a, w1b_ref[...], preferred_element_type=jnp.float32)
        + b1_ref[...],
        0.0,
    )
    out_ref[...] = (
        jnp.dot(t, w2_ref[...], preferred_element_type=jnp.float32)
        + b2_ref[...]
        + xv
    )


def _node_mlp(x, agg2, w1a, w1b, b1, w2, b2):
    nblk = N // NB_NODE
    off = N // NB_NODE
    full = lambda shape: pl.BlockSpec(shape, lambda i: (0, 0))
    return pl.pallas_call(
        _node_body,
        grid=(nblk,),
        in_specs=[
            pl.BlockSpec((NB_NODE, D), lambda i: (i, 0)),
            pl.BlockSpec((NB_NODE, H), lambda i: (i, 0)),
            pl.BlockSpec((NB_NODE, H), lambda i: (i + off, 0)),
            full((D, H)),
            full((H, H)),
            full((1, H)),
            full((H, D)),
            full((1, D)),
        ],
        out_specs=pl.BlockSpec((NB_NODE, D), lambda i: (i, 0)),
        out_shape=jax.ShapeDtypeStruct((N, D), jnp.float32),
    )(x, agg2, agg2, w1a, w1b, b1, w2, b2)


def kernel(x, edge_index, edge_attr, W_e1, b_e1, W_e2, b_e2,
           W_n1, b_n1, W_n2, b_n2):
    row = edge_index[0].astype(jnp.int32)
    col = edge_index[1].astype(jnp.int32)

    wa = W_e1[:D]
    wb = W_e1[D : 2 * D]
    wc = W_e1[2 * D :]
    be1 = b_e1.reshape(1, H)
    be2 = b_e2.reshape(1, H)
    w1a = W_n1[:D]
    w1b = W_n1[D:]
    bn1 = b_n1.reshape(1, H)
    bn2 = b_n2.reshape(1, D)

    row3s = row[: NW * CPW * CH].reshape(NW, CPW, CH)
    xa, xb = _node_tables(x, wa, wb, be1)
    g = _gather_add(xa, xb, row, col)
    ef = _edge_mlp(g, edge_attr, wc, W_e2, be2)
    agg2 = _segment_sum(ef, row3s, row)
    out = _node_mlp(x, agg2, w1a, w1b, bn1, W_n2, bn2)
    return (out, ef)


# speedup vs baseline: 1.0042x; 1.0042x over previous
import functools

import jax
import jax.numpy as jnp
from jax import lax
from jax.experimental import pallas as pl
from jax.experimental.pallas import tpu as pltpu
from jax.experimental.pallas import tpu_sc as plsc

N, E, D, DE, H = 10000, 320000, 128, 16, 128

NC, NS = 2, 16
NW = NC * NS
CH = 128
NCHUNK = E // CH
T_FULL = 632
T_LAST = N - (NS - 1) * T_FULL

NB_NODE = 2000
NB_EDGE = 16000


def _pre_body(x_ref, wa_ref, wb_ref, be1_ref, xa_ref, xb_ref):
    xv = x_ref[...]
    xa_ref[...] = (
        jnp.dot(xv, wa_ref[...], preferred_element_type=jnp.float32)
        + be1_ref[...]
    )
    xb_ref[...] = jnp.dot(xv, wb_ref[...], preferred_element_type=jnp.float32)


def _node_tables(x, wa, wb, be1):
    nblk = N // NB_NODE
    full = lambda shape: pl.BlockSpec(shape, lambda i: (0, 0))
    return pl.pallas_call(
        _pre_body,
        grid=(nblk,),
        in_specs=[
            pl.BlockSpec((NB_NODE, D), lambda i: (i, 0)),
            full((D, H)),
            full((D, H)),
            full((1, H)),
        ],
        out_specs=[
            pl.BlockSpec((NB_NODE, H), lambda i: (i, 0)),
            pl.BlockSpec((NB_NODE, H), lambda i: (i, 0)),
        ],
        out_shape=[
            jax.ShapeDtypeStruct((N, H), jnp.float32),
            jax.ShapeDtypeStruct((N, H), jnp.float32),
        ],
    )(x, wa, wb, be1)


CPW = NCHUNK // NW
NTAIL = NCHUNK - CPW * NW

NIT_G = CPW // 2


def _gather_body(xa_hbm, xb_hbm, row_hbm, col_hbm, g_hbm,
                 idx_r, idx_c, abuf, bbuf, obuf,
                 sg0, sg1, si0, si1, sw0, sw1):
    sg = [sg0, sg1]
    si = [si0, si1]
    sw = [sw0, sw1]
    wid = lax.axis_index("s") * NC + lax.axis_index("c")
    start = wid * CPW

    for b in range(2):
        c0 = start + b
        pltpu.sync_copy(row_hbm.at[pl.ds(c0 * CH, CH)], idx_r.at[b])
        pltpu.sync_copy(col_hbm.at[pl.ds(c0 * CH, CH)], idx_c.at[b])
        pltpu.async_copy(xa_hbm.at[idx_r.at[b]], abuf.at[b], sg[b])
        pltpu.async_copy(xb_hbm.at[idx_c.at[b]], bbuf.at[b], sg[b])

    def slot(i, b):
        c = start + 2 * i + b
        pltpu.make_async_copy(
            xa_hbm.at[idx_r.at[b]], abuf.at[b], sg[b]).wait()
        pltpu.make_async_copy(
            xb_hbm.at[idx_c.at[b]], bbuf.at[b], sg[b]).wait()

        @pl.when(i < NIT_G - 1)
        def _():
            cn = c + 2
            pltpu.async_copy(row_hbm.at[pl.ds(cn * CH, CH)], idx_r.at[b],
                             si[b])
            pltpu.async_copy(col_hbm.at[pl.ds(cn * CH, CH)], idx_c.at[b],
                             si[b])

        @pl.when(i > 0)
        def _():
            pltpu.make_async_copy(
                obuf.at[b], g_hbm.at[pl.ds((c - 2) * CH, CH)], sw[b]).wait()

        @plsc.parallel_loop(0, CH, 1, unroll=8)
        def _(r):
            for j in range(H // 16):
                sl = pl.ds(j * 16, 16)
                obuf[b, r, sl] = abuf[b, r, sl] + bbuf[b, r, sl]

        pltpu.async_copy(obuf.at[b], g_hbm.at[pl.ds(c * CH, CH)], sw[b])

        @pl.when(i < NIT_G - 1)
        def _():
            cn = c + 2
            pltpu.make_async_copy(row_hbm.at[pl.ds(cn * CH, CH)],
                                  idx_r.at[b], si[b]).wait()
            pltpu.make_async_copy(col_hbm.at[pl.ds(cn * CH, CH)],
                                  idx_c.at[b], si[b]).wait()
            pltpu.async_copy(xa_hbm.at[idx_r.at[b]], abuf.at[b], sg[b])
            pltpu.async_copy(xb_hbm.at[idx_c.at[b]], bbuf.at[b], sg[b])

    def it(i, carry):
        slot(i, 0)
        slot(i, 1)
        return carry

    lax.fori_loop(0, NIT_G, it, 0)

    for b in range(2):
        cl = start + 2 * (NIT_G - 1) + b
        pltpu.make_async_copy(
            obuf.at[b], g_hbm.at[pl.ds(cl * CH, CH)], sw[b]).wait()

    @pl.when(wid < NTAIL)
    def _():
        ct = NW * CPW + wid
        pltpu.sync_copy(row_hbm.at[pl.ds(ct * CH, CH)], idx_r.at[0])
        pltpu.sync_copy(col_hbm.at[pl.ds(ct * CH, CH)], idx_c.at[0])
        pltpu.async_copy(xa_hbm.at[idx_r.at[0]], abuf.at[0], sg[0]).wait()
        pltpu.async_copy(xb_hbm.at[idx_c.at[0]], bbuf.at[0], sg[0]).wait()

        def add_row(r, carry):
            for j in range(H // 16):
                sl = pl.ds(j * 16, 16)
                obuf[0, r, sl] = abuf[0, r, sl] + bbuf[0, r, sl]
            return carry

        lax.fori_loop(0, CH, add_row, 0)
        pltpu.sync_copy(obuf.at[0], g_hbm.at[pl.ds(ct * CH, CH)])


def _gather_add(xa, xb, row, col):
    mesh = plsc.VectorSubcoreMesh(core_axis_name="c", subcore_axis_name="s")
    fn = functools.partial(
        pl.kernel,
        mesh=mesh,
        out_type=jax.ShapeDtypeStruct((E, H), jnp.float32),
        scratch_types=[
            pltpu.VMEM((2, CH), jnp.int32),
            pltpu.VMEM((2, CH), jnp.int32),
            pltpu.VMEM((2, CH, H), jnp.float32),
            pltpu.VMEM((2, CH, H), jnp.float32),
            pltpu.VMEM((2, CH, H), jnp.float32),
            pltpu.SemaphoreType.DMA,
            pltpu.SemaphoreType.DMA,
            pltpu.SemaphoreType.DMA,
            pltpu.SemaphoreType.DMA,
            pltpu.SemaphoreType.DMA,
            pltpu.SemaphoreType.DMA,
        ],
    )(_gather_body)
    return fn(xa, xb, row, col)


def _edge_body(g_ref, ea_ref, wc_ref, w2_ref, b2_ref, ef_ref):
    h = jnp.maximum(
        g_ref[...]
        + jnp.dot(ea_ref[...], wc_ref[...], preferred_element_type=jnp.float32),
        0.0,
    )
    ef_ref[...] = jnp.maximum(
        jnp.dot(h, w2_ref[...], preferred_element_type=jnp.float32)
        + b2_ref[...],
        0.0,
    )


def _edge_mlp(g, ea, wc, w2, b2):
    nblk = E // NB_EDGE
    full = lambda shape: pl.BlockSpec(shape, lambda i: (0, 0))
    return pl.pallas_call(
        _edge_body,
        grid=(nblk,),
        in_specs=[
            pl.BlockSpec((NB_EDGE, H), lambda i: (i, 0)),
            pl.BlockSpec((NB_EDGE, DE), lambda i: (i, 0)),
            full((DE, H)),
            full((H, H)),
            full((1, H)),
        ],
        out_specs=pl.BlockSpec((NB_EDGE, H), lambda i: (i, 0)),
        out_shape=jax.ShapeDtypeStruct((E, H), jnp.float32),
    )(g, ea, wc, w2, b2)


NIT_S = CPW // 3


def _scatter_body(ef_hbm, row1_hbm, agg_hbm, idx_s, ebuf, accum,
                  sf0, sf1, sf2, ss0, ss1, ss2):
    sf = [sf0, sf1, sf2]
    ss = [ss0, ss1, ss2]
    c_id = lax.axis_index("c")
    s_id = lax.axis_index("s")
    wid = s_id * NC + c_id

    zero = jnp.zeros((16,), jnp.float32)

    def zrow(r, carry):
        for j in range(H // 16):
            ebuf[0, r, pl.ds(j * 16, 16)] = zero
        return carry

    lax.fori_loop(0, CH, zrow, 0)
    base_rows = s_id * T_FULL

    @pl.when(s_id < NS - 1)
    def _():
        for k in range(4):
            pltpu.sync_copy(
                ebuf.at[0, pl.ds(0, CH)],
                accum.at[pl.ds(base_rows + k * CH, CH)],
            )
        pltpu.sync_copy(
            ebuf.at[0, pl.ds(0, T_FULL - 4 * CH)],
            accum.at[pl.ds(base_rows + 4 * CH, T_FULL - 4 * CH)],
        )

    @pl.when(s_id == NS - 1)
    def _():
        for k in range(4):
            pltpu.sync_copy(
                ebuf.at[0, pl.ds(0, CH)],
                accum.at[pl.ds(base_rows + k * CH, CH)],
            )
        pltpu.sync_copy(
            ebuf.at[0, pl.ds(0, T_LAST - 4 * CH)],
            accum.at[pl.ds(base_rows + 4 * CH, T_LAST - 4 * CH)],
        )

    plsc.subcore_barrier()

    start = wid * CPW

    for b in range(3):
        c0 = start + b
        pltpu.async_copy(row1_hbm.at[pl.ds(c0 * CH, CH)], idx_s.at[b], sf[b])
        pltpu.async_copy(ef_hbm.at[pl.ds(c0 * CH, CH)], ebuf.at[b], sf[b])

    def slot(i, b):
        c = start + 3 * i + b
        k = 3 * i + b
        pltpu.make_async_copy(
            row1_hbm.at[pl.ds(c * CH, CH)], idx_s.at[b], sf[b]).wait()
        pltpu.make_async_copy(
            ef_hbm.at[pl.ds(c * CH, CH)], ebuf.at[b], sf[b]).wait()
        pltpu.async_copy(ebuf.at[b], accum.at[idx_s.at[b]], ss[b], add=True)

        br = (b + 1) % 3

        @pl.when((k >= 2) & (k <= CPW - 2))
        def _():
            pltpu.make_async_copy(
                ebuf.at[br], accum.at[idx_s.at[br]], ss[br]).wait()
            cn = c + 1
            pltpu.async_copy(row1_hbm.at[pl.ds(cn * CH, CH)], idx_s.at[br],
                             sf[br])
            pltpu.async_copy(ef_hbm.at[pl.ds(cn * CH, CH)], ebuf.at[br],
                             sf[br])

    def it(i, carry):
        slot(i, 0)
        slot(i, 1)
        slot(i, 2)
        return carry

    lax.fori_loop(0, NIT_S, it, 0)

    for b in range(3):
        pltpu.make_async_copy(
            ebuf.at[b], accum.at[idx_s.at[b]], ss[b]).wait()

    @pl.when(wid < NTAIL)
    def _():
        ct = NW * CPW + wid
        pltpu.sync_copy(row1_hbm.at[pl.ds(ct * CH, CH)], idx_s.at[0])
        pltpu.sync_copy(ef_hbm.at[pl.ds(ct * CH, CH)], ebuf.at[0])
        pltpu.sync_copy(ebuf.at[0], accum.at[idx_s.at[0]], add=True)

    plsc.subcore_barrier()

    @pl.when(s_id < NS - 1)
    def _():
        pltpu.sync_copy(
            accum.at[pl.ds(base_rows, T_FULL)],
            agg_hbm.at[pl.ds(c_id * N + base_rows, T_FULL)],
        )

    @pl.when(s_id == NS - 1)
    def _():
        pltpu.sync_copy(
            accum.at[pl.ds(base_rows, T_LAST)],
            agg_hbm.at[pl.ds(c_id * N + base_rows, T_LAST)],
        )


def _segment_sum(ef, row1):
    mesh = plsc.VectorSubcoreMesh(core_axis_name="c", subcore_axis_name="s")
    fn = functools.partial(
        pl.kernel,
        mesh=mesh,
        out_type=jax.ShapeDtypeStruct((NC * N, H), jnp.float32),
        scratch_types=[
            pltpu.VMEM((3, CH), jnp.int32),
            pltpu.VMEM((3, CH, H), jnp.float32),
            pltpu.VMEM_SHARED((N, H), jnp.float32),
            pltpu.SemaphoreType.DMA,
            pltpu.SemaphoreType.DMA,
            pltpu.SemaphoreType.DMA,
            pltpu.SemaphoreType.DMA,
            pltpu.SemaphoreType.DMA,
            pltpu.SemaphoreType.DMA,
        ],
    )(_scatter_body)
    return fn(ef, row1)


def _node_body(x_ref, a0_ref, a1_ref, w1a_ref, w1b_ref, b1_ref, w2_ref,
               b2_ref, out_ref):
    xv = x_ref[...]
    a = a0_ref[...] + a1_ref[...]
    t = jnp.maximum(
        jnp.dot(xv, w1a_ref[...], preferred_element_type=jnp.float32)
        + jnp.dot(a, w1b_ref[...], preferred_element_type=jnp.float32)
        + b1_ref[...],
        0.0,
    )
    out_ref[...] = (
        jnp.dot(t, w2_ref[...], preferred_element_type=jnp.float32)
        + b2_ref[...]
        + xv
    )


def _node_mlp(x, agg2, w1a, w1b, b1, w2, b2):
    nblk = N // NB_NODE
    off = N // NB_NODE
    full = lambda shape: pl.BlockSpec(shape, lambda i: (0, 0))
    return pl.pallas_call(
        _node_body,
        grid=(nblk,),
        in_specs=[
            pl.BlockSpec((NB_NODE, D), lambda i: (i, 0)),
            pl.BlockSpec((NB_NODE, H), lambda i: (i, 0)),
            pl.BlockSpec((NB_NODE, H), lambda i: (i + off, 0)),
            full((D, H)),
            full((H, H)),
            full((1, H)),
            full((H, D)),
            full((1, D)),
        ],
        out_specs=pl.BlockSpec((NB_NODE, D), lambda i: (i, 0)),
        out_shape=jax.ShapeDtypeStruct((N, D), jnp.float32),
    )(x, agg2, agg2, w1a, w1b, b1, w2, b2)


def kernel(x, edge_index, edge_attr, W_e1, b_e1, W_e2, b_e2,
           W_n1, b_n1, W_n2, b_n2):
    row = edge_index[0].astype(jnp.int32)
    col = edge_index[1].astype(jnp.int32)

    wa = W_e1[:D]
    wb = W_e1[D : 2 * D]
    wc = W_e1[2 * D :]
    be1 = b_e1.reshape(1, H)
    be2 = b_e2.reshape(1, H)
    w1a = W_n1[:D]
    w1b = W_n1[D:]
    bn1 = b_n1.reshape(1, H)
    bn2 = b_n2.reshape(1, D)

    xa, xb = _node_tables(x, wa, wb, be1)
    g = _gather_add(xa, xb, row, col)
    ef = _edge_mlp(g, edge_attr, wc, W_e2, be2)
    agg2 = _segment_sum(ef, row)
    out = _node_mlp(x, agg2, w1a, w1b, bn1, W_n2, bn2)
    return (out, ef)
